# Initial kernel scaffold; baseline (speedup 1.0000x reference)
import jax
import jax.numpy as jnp
from jax.experimental import pallas as pl


def kernel(x_dyn, W1, b1, Wq, Wk):
    raise NotImplementedError("write your pallas kernel here")


# speedup vs baseline: 5.6454x; 5.6454x over previous
import jax
import jax.numpy as jnp
import numpy as np
from jax.experimental import pallas as pl

B, N, T, D_DYN = 2, 4096, 12, 2
HID = 64
HEADS = 4
DH = HID // HEADS
TOPK = 10
KPAD = 128
R1 = 256
R2 = 256
NEG = float("-inf")


def _topk_kernel(a_ref, vals_ref, idx_ref):
    acc = a_ref[0]
    col = jax.lax.broadcasted_iota(jnp.int32, (R1, N), 1)
    lane = jax.lax.broadcasted_iota(jnp.int32, (R1, KPAD), 1)
    vals = jnp.zeros((R1, KPAD), dtype=jnp.float32)
    idxs = jnp.zeros((R1, KPAD), dtype=jnp.int32)
    work = acc
    for kk in range(TOPK):
        m = jnp.max(work, axis=-1, keepdims=True)
        am = jnp.min(jnp.where(work == m, col, N), axis=-1, keepdims=True)
        vals = jnp.where(lane == kk, m, vals)
        idxs = jnp.where(lane == kk, am, idxs)
        work = jnp.where(col == am, NEG, work)
    vals_ref[0] = vals
    idx_ref[0] = idxs


def _transpose_kernel(vals_ref, idx_ref, vals_t_ref, idx_t_ref):
    vals_t_ref[0] = vals_ref[0].T
    idx_t_ref[0] = idx_ref[0].T


def _scatter_kernel(vals_ref, idx_ref, vals_t_ref, idx_t_ref, a_ref, l_ref):
    i = pl.program_id(1)
    r0 = i * R2
    col = jax.lax.broadcasted_iota(jnp.int32, (R2, N), 1)
    rowv = r0 + jax.lax.broadcasted_iota(jnp.int32, (R2, 1), 0)
    acc = jnp.zeros((R2, N), dtype=jnp.float32)
    for kk in range(TOPK):
        ri = idx_ref[0, :, kk : kk + 1]
        rv = vals_ref[0, :, kk : kk + 1]
        acc = acc + jnp.where(col == ri, rv, 0.0)
        ci = idx_t_ref[0, kk : kk + 1, :]
        cv = vals_t_ref[0, kk : kk + 1, :]
        acc = acc + jnp.where(ci == rowv, cv, 0.0)
    acc = acc * 0.5
    deg = jnp.sum(acc, axis=-1, keepdims=True)
    diag = col == rowv
    a_ref[0] = acc
    l_ref[0] = jnp.where(diag, deg, 0.0) - acc


def kernel(x_dyn, W1, b1, Wq, Wk):
    h = jax.nn.relu(jnp.einsum('bntd,dh->bnth', x_dyn, W1) + b1)
    h = h.mean(axis=2)
    q = (h @ Wq).reshape(B, N, HEADS, DH).transpose(0, 2, 1, 3)
    k = (h @ Wk).reshape(B, N, HEADS, DH).transpose(0, 2, 1, 3)
    scores = jnp.einsum('bhnd,bhmd->bhnm', q, k) / np.sqrt(DH)
    attn = jax.nn.softmax(scores, axis=-1)
    A_learn = attn.mean(axis=1)

    vals, idxs = pl.pallas_call(
        _topk_kernel,
        grid=(B, N // R1),
        in_specs=[pl.BlockSpec((1, R1, N), lambda b, i: (b, i, 0))],
        out_specs=[
            pl.BlockSpec((1, R1, KPAD), lambda b, i: (b, i, 0)),
            pl.BlockSpec((1, R1, KPAD), lambda b, i: (b, i, 0)),
        ],
        out_shape=[
            jax.ShapeDtypeStruct((B, N, KPAD), jnp.float32),
            jax.ShapeDtypeStruct((B, N, KPAD), jnp.int32),
        ],
    )(A_learn)

    vals_t, idx_t = pl.pallas_call(
        _transpose_kernel,
        grid=(B,),
        in_specs=[
            pl.BlockSpec((1, N, KPAD), lambda b: (b, 0, 0)),
            pl.BlockSpec((1, N, KPAD), lambda b: (b, 0, 0)),
        ],
        out_specs=[
            pl.BlockSpec((1, KPAD, N), lambda b: (b, 0, 0)),
            pl.BlockSpec((1, KPAD, N), lambda b: (b, 0, 0)),
        ],
        out_shape=[
            jax.ShapeDtypeStruct((B, KPAD, N), jnp.float32),
            jax.ShapeDtypeStruct((B, KPAD, N), jnp.int32),
        ],
    )(vals, idxs)

    A_final, L = pl.pallas_call(
        _scatter_kernel,
        grid=(B, N // R2),
        in_specs=[
            pl.BlockSpec((1, R2, KPAD), lambda b, i: (b, i, 0)),
            pl.BlockSpec((1, R2, KPAD), lambda b, i: (b, i, 0)),
            pl.BlockSpec((1, KPAD, N), lambda b, i: (b, 0, 0)),
            pl.BlockSpec((1, KPAD, N), lambda b, i: (b, 0, 0)),
        ],
        out_specs=[
            pl.BlockSpec((1, R2, N), lambda b, i: (b, i, 0)),
            pl.BlockSpec((1, R2, N), lambda b, i: (b, i, 0)),
        ],
        out_shape=[
            jax.ShapeDtypeStruct((B, N, N), jnp.float32),
            jax.ShapeDtypeStruct((B, N, N), jnp.float32),
        ],
    )(vals, idxs, vals_t, idx_t)

    return (L, A_final)
